# baseline (speedup 1.0000x reference)
import functools

import jax
import jax.numpy as jnp
from jax import lax
from jax.experimental import pallas as pl
from jax.experimental.pallas import tpu as pltpu
from jax.experimental.pallas import tpu_sc as plsc

NUM_TOKENS = 1000000
DIM_MODEL = 64
BATCH = 4096
SEQ = 200

NC = 2
NS = 16
NW = NC * NS

N = BATCH * SEQ
N_PER_W = N // NW
CHUNK = 128
K = 1
GROUP = CHUNK * K
GROUPS = N_PER_W // GROUP
NSLOT = 4
BG = BATCH // 128
UNITS_PER_W = N_PER_W // CHUNK


def _gather_sc(table, idx):
    mesh = plsc.VectorSubcoreMesh(core_axis_name="c", subcore_axis_name="s")

    @functools.partial(
        pl.kernel,
        mesh=mesh,
        out_type=jax.ShapeDtypeStruct(
            (SEQ, DIM_MODEL // 8, BG, 8, 128), jnp.float32
        ),
        scratch_types=[
            pltpu.VMEM((N_PER_W,), jnp.int32),
            pltpu.VMEM((NSLOT, GROUP, DIM_MODEL), jnp.float32),
            pltpu.VMEM((NSLOT, 8, 8, 129), jnp.float32),
            [pltpu.SemaphoreType.DMA] * NSLOT,
            [pltpu.SemaphoreType.DMA] * NSLOT,
        ],
        compiler_params=pltpu.CompilerParams(
            use_tc_tiling_on_sc=False, needs_layout_passes=False
        ),
    )
    def k(table_hbm, idx_hbm, out_hbm, idx_v, rows_v, t_v, gsems, ssems):
        wid = lax.axis_index("s") * NC + lax.axis_index("c")
        base = wid * N_PER_W
        u_base = wid * UNITS_PER_W
        pltpu.sync_copy(idx_hbm.at[pl.ds(base, N_PER_W)], idx_v)

        def issue_gathers(g, s):
            off = g * GROUP
            pltpu.async_copy(
                table_hbm.at[idx_v.at[pl.ds(off, CHUNK)]],
                rows_v.at[s],
                gsems[s],
            )

        def wait_gathers(g, s):
            off = g * GROUP
            pltpu.make_async_copy(
                table_hbm.at[idx_v.at[pl.ds(off, CHUNK)]],
                rows_v.at[s],
                gsems[s],
            ).wait()

        iota16 = lax.iota(jnp.int32, 16)
        fg_base = lax.shift_right_logical(iota16, 3)
        fi_const = lax.bitwise_and(iota16, 7)

        def transpose_group(g, s, ts):
            rows = rows_v.at[s]
            tv = t_v.at[ts]
            fg_vecs = [fg_base + fb * 2 for fb in range(4)]

            @plsc.parallel_loop(0, CHUNK, step=8)
            def tb_body(t0):
                vals = [
                    (dt, fb, rows[t0 + dt, pl.ds(fb * 16, 16)])
                    for dt in range(8)
                    for fb in range(4)
                ]
                t_vecs = [jnp.full((16,), t0 + dt, jnp.int32) for dt in range(8)]
                for dt, fb, v in vals:
                    plsc.store_scatter(
                        tv, [fg_vecs[fb], fi_const, t_vecs[dt]], v
                    )

        def unit_dst(g, kk):
            u = u_base + g * K + kk
            s_row = lax.shift_right_logical(u, 5)
            bg = lax.bitwise_and(u, BG - 1)
            return out_hbm.at[s_row, :, bg]

        def issue_stores(g, ts):
            pltpu.async_copy(
                t_v.at[ts, :, :, pl.ds(0, 128)], unit_dst(g, 0), ssems[ts]
            )

        def wait_stores(g, ts):
            pltpu.make_async_copy(
                t_v.at[ts, :, :, pl.ds(0, 128)], unit_dst(g, 0), ssems[ts]
            ).wait()

        for g0 in range(NSLOT):
            issue_gathers(g0, g0)

        def quad_body(gq, carry):
            for h in range(NSLOT):
                g = NSLOT * gq + h
                wait_gathers(g, h)

                @pl.when(g >= NSLOT)
                def _():
                    wait_stores(g - NSLOT, h)

                transpose_group(g, h, h)
                issue_stores(g, h)

                @pl.when(g + NSLOT < GROUPS)
                def _():
                    issue_gathers(g + NSLOT, h)

            return carry

        lax.fori_loop(0, GROUPS // NSLOT, quad_body, 0)

        for g in range(GROUPS - NSLOT, GROUPS):
            wait_stores(g, g % NSLOT)

    return k(table, idx)


def kernel(src, embedding):
    idx = jnp.transpose(src).reshape(-1).astype(jnp.int32)
    out5 = _gather_sc(embedding, idx)
    return jnp.transpose(out5, (2, 4, 0, 1, 3)).reshape(BATCH, SEQ, DIM_MODEL)


# speedup vs baseline: 1.3895x; 1.3895x over previous
import functools

import jax
import jax.numpy as jnp
from jax import lax
from jax.experimental import pallas as pl
from jax.experimental.pallas import tpu as pltpu
from jax.experimental.pallas import tpu_sc as plsc

NUM_TOKENS = 1000000
DIM_MODEL = 64
BATCH = 4096
SEQ = 200

NC = 2
NS = 16
NW = NC * NS

N = BATCH * SEQ
N_PER_W = N // NW
CHUNK = 128
K = 1
GROUP = CHUNK * K
GROUPS = N_PER_W // GROUP
NSLOT = 4
BG = BATCH // 128
UNITS_PER_W = N_PER_W // CHUNK


def _gather_sc(table, idx):
    mesh = plsc.VectorSubcoreMesh(core_axis_name="c", subcore_axis_name="s")

    @functools.partial(
        pl.kernel,
        mesh=mesh,
        out_type=jax.ShapeDtypeStruct(
            (SEQ, DIM_MODEL // 8, BG, 8, 128), jnp.float32
        ),
        scratch_types=[
            pltpu.VMEM((N_PER_W,), jnp.int32),
            pltpu.VMEM((NSLOT, GROUP, DIM_MODEL), jnp.float32),
            pltpu.VMEM((NSLOT, 8, 8, 129), jnp.float32),
            [pltpu.SemaphoreType.DMA] * NSLOT,
            [pltpu.SemaphoreType.DMA] * NSLOT,
        ],
        compiler_params=pltpu.CompilerParams(
            use_tc_tiling_on_sc=False, needs_layout_passes=False
        ),
    )
    def k(table_hbm, idx_hbm, out_hbm, idx_v, rows_v, t_v, gsems, ssems):
        wid = lax.axis_index("s") * NC + lax.axis_index("c")
        base = wid * N_PER_W
        u_base = wid * UNITS_PER_W
        pltpu.sync_copy(idx_hbm.at[pl.ds(base, N_PER_W)], idx_v)

        def issue_gathers(g, s):
            off = g * GROUP
            pltpu.async_copy(
                table_hbm.at[idx_v.at[pl.ds(off, CHUNK)]],
                rows_v.at[s],
                gsems[s],
            )

        def wait_gathers(g, s):
            off = g * GROUP
            pltpu.make_async_copy(
                table_hbm.at[idx_v.at[pl.ds(off, CHUNK)]],
                rows_v.at[s],
                gsems[s],
            ).wait()

        iota16 = lax.iota(jnp.int32, 16)
        fg_base = lax.shift_right_logical(iota16, 3)
        fi_const = lax.bitwise_and(iota16, 7)

        def transpose_group(g, s, ts):
            rows = rows_v.at[s]
            tv = t_v.at[ts]
            fg_vecs = [fg_base + fb * 2 for fb in range(4)]

            def tb_body(tb, carry):
                t0 = tb * 8
                vals = [
                    (dt, fb, rows[t0 + dt, pl.ds(fb * 16, 16)])
                    for dt in range(8)
                    for fb in range(4)
                ]
                t_vecs = [jnp.full((16,), t0 + dt, jnp.int32) for dt in range(8)]
                for dt, fb, v in vals:
                    plsc.store_scatter(
                        tv, [fg_vecs[fb], fi_const, t_vecs[dt]], v
                    )
                return carry

            lax.fori_loop(0, CHUNK // 8, tb_body, 0)

        def unit_dst(g, kk):
            u = u_base + g * K + kk
            s_row = lax.shift_right_logical(u, 5)
            bg = lax.bitwise_and(u, BG - 1)
            return out_hbm.at[s_row, :, bg]

        def issue_stores(g, ts):
            pltpu.async_copy(
                t_v.at[ts, :, :, pl.ds(0, 128)], unit_dst(g, 0), ssems[ts]
            )

        def wait_stores(g, ts):
            pltpu.make_async_copy(
                t_v.at[ts, :, :, pl.ds(0, 128)], unit_dst(g, 0), ssems[ts]
            ).wait()

        for g0 in range(NSLOT):
            issue_gathers(g0, g0)

        def quad_body(gq, carry):
            for h in range(NSLOT):
                g = NSLOT * gq + h
                wait_gathers(g, h)

                @pl.when(g >= NSLOT)
                def _():
                    wait_stores(g - NSLOT, h)

                transpose_group(g, h, h)
                issue_stores(g, h)

                @pl.when(g + NSLOT < GROUPS)
                def _():
                    issue_gathers(g + NSLOT, h)

            return carry

        lax.fori_loop(0, GROUPS // NSLOT, quad_body, 0)

        for g in range(GROUPS - NSLOT, GROUPS):
            wait_stores(g, g % NSLOT)

    return k(table, idx)


def kernel(src, embedding):
    idx = jnp.transpose(src).reshape(-1).astype(jnp.int32)
    out5 = _gather_sc(embedding, idx)
    return jnp.transpose(out5, (2, 4, 0, 1, 3)).reshape(BATCH, SEQ, DIM_MODEL)
